# Initial kernel scaffold; baseline (speedup 1.0000x reference)
import jax
import jax.numpy as jnp
from jax.experimental import pallas as pl


def kernel(x, enc_W, enc_b, dec_W, gater_W, gater_b, b_dec, b_gate):
    raise NotImplementedError("write your pallas kernel here")


# speedup vs baseline: 10.6151x; 10.6151x over previous
import functools

import jax
import jax.numpy as jnp
from jax.experimental import pallas as pl
from jax.experimental.pallas import tpu as pltpu

ACT_DIM = 768
DICT = 24576
EXPERTS = 16
E = 2
K = 32
EXPERT_DICT = 1536
TOK_TILE = 256


def _topk_mask(z):
    zi = jax.lax.bitcast_convert_type(z, jnp.int32)
    rows = z.shape[0]
    t = jnp.zeros((rows, 1), jnp.int32)
    for bit in range(30, -1, -1):
        cand = t | (1 << bit)
        cnt = jnp.sum((zi >= cand).astype(jnp.int32), axis=1, keepdims=True)
        t = jnp.where(cnt >= K, cand, t)
    gt = zi > t
    n_gt = jnp.sum(gt.astype(jnp.int32), axis=1, keepdims=True)
    m = K - n_gt
    eq = zi == t
    lane = jax.lax.broadcasted_iota(jnp.int32, z.shape, 1)
    c = jnp.zeros((rows, 1), jnp.int32)
    for bit in range(10, -1, -1):
        cand = c | (1 << bit)
        f = jnp.sum((eq & (lane < cand)).astype(jnp.int32), axis=1, keepdims=True)
        c = jnp.where(f < m, cand, c)
    return gt | (eq & (lane <= c))


def _moe_kernel(x_ref, encW_ref, encb_ref, decW_ref, gaterW_ref, gaterb_ref,
                bdec_ref, bgate_ref, out_ref, wf_ref):
    e = pl.program_id(1)
    x = x_ref[...]
    tile = x.shape[0]
    col16 = jax.lax.broadcasted_iota(jnp.int32, (tile, EXPERTS), 1)

    @pl.when(e == 0)
    def _gate():
        gl = jax.lax.dot_general(
            x - bgate_ref[...], gaterW_ref[...],
            (((1,), (1,)), ((), ())), preferred_element_type=jnp.float32,
        ) + gaterb_ref[...]
        probs = jnp.maximum(gl, 0.0)
        m1 = jnp.max(probs, axis=1, keepdims=True)
        i1 = jnp.min(jnp.where(probs == m1, col16, EXPERTS), axis=1, keepdims=True)
        probs2 = jnp.where(col16 == i1, -1.0, probs)
        m2 = jnp.max(probs2, axis=1, keepdims=True)
        i2 = jnp.min(jnp.where(probs2 == m2, col16, EXPERTS), axis=1, keepdims=True)
        in_top0 = ((i1 == 0) | (i2 == 0)).astype(jnp.float32)
        in_top1 = ((i1 == 1) | (i2 == 1)).astype(jnp.float32)
        w0 = probs[:, 0:1] * in_top0
        w1 = probs[:, 1:2] * in_top1
        wf_ref[...] = (jnp.where(col16 == i1, w0, 0.0)
                       + jnp.where(col16 == i2, w1, 0.0))

    w = jnp.sum(wf_ref[...] * (col16 == e).astype(jnp.float32), axis=1,
                keepdims=True)

    xc = x - bdec_ref[...]
    z = jnp.maximum(
        jax.lax.dot_general(xc, encW_ref[0], (((1,), (1,)), ((), ())),
                            preferred_element_type=jnp.float32)
        + encb_ref[0], 0.0)
    keep = _topk_mask(z)
    mz = jnp.where(keep, z, 0.0)
    contrib = jax.lax.dot_general(mz, decW_ref[0], (((1,), (0,)), ((), ())),
                                  preferred_element_type=jnp.float32) * w

    @pl.when(e == 0)
    def _init():
        out_ref[...] = bdec_ref[...] + contrib

    @pl.when(e != 0)
    def _acc():
        out_ref[...] += contrib


@jax.jit
def kernel(x, enc_W, enc_b, dec_W, gater_W, gater_b, b_dec, b_gate):
    n_tok = x.shape[0]
    grid = (n_tok // TOK_TILE, EXPERTS)
    return pl.pallas_call(
        _moe_kernel,
        grid=grid,
        in_specs=[
            pl.BlockSpec((TOK_TILE, ACT_DIM), lambda t, e: (t, 0)),
            pl.BlockSpec((1, EXPERT_DICT, ACT_DIM), lambda t, e: (e, 0, 0)),
            pl.BlockSpec((1, 1, EXPERT_DICT), lambda t, e: (e, 0, 0)),
            pl.BlockSpec((1, EXPERT_DICT, ACT_DIM), lambda t, e: (e, 0, 0)),
            pl.BlockSpec((EXPERTS, ACT_DIM), lambda t, e: (0, 0)),
            pl.BlockSpec((1, EXPERTS), lambda t, e: (0, 0)),
            pl.BlockSpec((1, ACT_DIM), lambda t, e: (0, 0)),
            pl.BlockSpec((1, ACT_DIM), lambda t, e: (0, 0)),
        ],
        out_specs=pl.BlockSpec((TOK_TILE, ACT_DIM), lambda t, e: (t, 0)),
        out_shape=jax.ShapeDtypeStruct((n_tok, ACT_DIM), x.dtype),
        scratch_shapes=[pltpu.VMEM((TOK_TILE, EXPERTS), jnp.float32)],
    )(x, enc_W, enc_b.reshape(EXPERTS, 1, EXPERT_DICT), dec_W, gater_W,
      gater_b.reshape(1, EXPERTS),
      b_dec.reshape(1, ACT_DIM), b_gate.reshape(1, ACT_DIM))


# speedup vs baseline: 26.0238x; 2.4516x over previous
import functools

import jax
import jax.numpy as jnp
from jax import lax
from jax.experimental import pallas as pl
from jax.experimental.pallas import tpu as pltpu
from jax.experimental.pallas import tpu_sc as plsc

ACT_DIM = 768
EXPERTS = 16
K = 32
EXPERT_DICT = 1536
N_TOK = 2048
TP = 64
N_PAIRS = 2 * N_TOK
NT = 84
P_PAD = NT * TP
NW = 32
DISP_PER_W = P_PAD // NW
COMB_PER_W = N_TOK // NW


def _gate_kernel(x_ref, gaterW_ref, gaterb_ref, bgate_ref, idx_ref, w_ref):
    x = x_ref[...]
    n = x.shape[0]
    col = lax.broadcasted_iota(jnp.int32, (n, EXPERTS), 1)
    gl = lax.dot_general(x - bgate_ref[...], gaterW_ref[...],
                         (((1,), (1,)), ((), ())),
                         preferred_element_type=jnp.float32) + gaterb_ref[...]
    probs = jnp.maximum(gl, 0.0)
    m1 = jnp.max(probs, axis=1, keepdims=True)
    i1 = jnp.min(jnp.where(probs == m1, col, EXPERTS), axis=1, keepdims=True)
    probs2 = jnp.where(col == i1, -1.0, probs)
    m2 = jnp.max(probs2, axis=1, keepdims=True)
    i2 = jnp.min(jnp.where(probs2 == m2, col, EXPERTS), axis=1, keepdims=True)
    in_top0 = ((i1 == 0) | (i2 == 0)).astype(jnp.float32)
    in_top1 = ((i1 == 1) | (i2 == 1)).astype(jnp.float32)
    idx_ref[...] = jnp.concatenate([i1, i2], axis=1)
    w_ref[...] = jnp.concatenate([probs[:, 0:1] * in_top0,
                                  probs[:, 1:2] * in_top1], axis=1)


def _sc_dispatch(x_hbm, tok_hbm, xs_hbm, idx_v, rows_v, sem):
    wid = lax.axis_index("s") * 2 + lax.axis_index("c")
    base = wid * DISP_PER_W
    pltpu.sync_copy(tok_hbm.at[pl.ds(base, DISP_PER_W)], idx_v)
    c0 = pltpu.async_copy(x_hbm.at[idx_v.at[pl.ds(0, 88)]],
                          rows_v.at[pl.ds(0, 88)], sem)
    c1 = pltpu.async_copy(x_hbm.at[idx_v.at[pl.ds(88, 80)]],
                          rows_v.at[pl.ds(88, 80)], sem)
    c0.wait()
    c1.wait()
    pltpu.sync_copy(rows_v, xs_hbm.at[pl.ds(base, DISP_PER_W)])


def _topk_mask(z):
    zi = lax.bitcast_convert_type(z, jnp.int32)
    rows = z.shape[0]
    t = jnp.zeros((rows, 1), jnp.int32)
    for bit in range(30, -1, -1):
        cand = t | (1 << bit)
        cnt = jnp.sum((zi >= cand).astype(jnp.int32), axis=1, keepdims=True)
        t = jnp.where(cnt >= K, cand, t)
    gt = zi > t
    n_gt = jnp.sum(gt.astype(jnp.int32), axis=1, keepdims=True)
    m = K - n_gt
    eq = zi == t
    lane = lax.broadcasted_iota(jnp.int32, z.shape, 1)
    c = jnp.zeros((rows, 1), jnp.int32)
    for bit in range(10, -1, -1):
        cand = c | (1 << bit)
        f = jnp.sum((eq & (lane < cand)).astype(jnp.int32), axis=1,
                    keepdims=True)
        c = jnp.where(f < m, cand, c)
    return gt | (eq & (lane <= c))


def _grouped_kernel(eot_ref, hw_ref, xs_ref, encW_ref, encb_ref, decW_ref,
                    wp_ref, bdec_ref, outp_ref):
    i = pl.program_id(0)

    @pl.when(hw_ref[i] == 0)
    def _zero():
        outp_ref[...] = jnp.zeros_like(outp_ref)

    @pl.when(hw_ref[i] != 0)
    def _work():
        xc = xs_ref[...] - bdec_ref[...]
        z = jnp.maximum(
            lax.dot_general(xc, encW_ref[0], (((1,), (1,)), ((), ())),
                            preferred_element_type=jnp.float32)
            + encb_ref[0], 0.0)
        mz = jnp.where(_topk_mask(z), z, 0.0)
        outp_ref[...] = lax.dot_general(
            mz, decW_ref[0], (((1,), (0,)), ((), ())),
            preferred_element_type=jnp.float32) * wp_ref[...]


def _sc_combine(outp_hbm, pos0_hbm, pos1_hbm, g0_hbm, g1_hbm,
                p0_v, p1_v, r0_v, r1_v, sem):
    wid = lax.axis_index("s") * 2 + lax.axis_index("c")
    base = wid * COMB_PER_W
    pltpu.sync_copy(pos0_hbm.at[pl.ds(base, COMB_PER_W)], p0_v)
    pltpu.sync_copy(pos1_hbm.at[pl.ds(base, COMB_PER_W)], p1_v)
    c0 = pltpu.async_copy(outp_hbm.at[p0_v], r0_v, sem)
    c1 = pltpu.async_copy(outp_hbm.at[p1_v], r1_v, sem)
    c0.wait()
    c1.wait()
    pltpu.sync_copy(r0_v, g0_hbm.at[pl.ds(base, COMB_PER_W)])
    pltpu.sync_copy(r1_v, g1_hbm.at[pl.ds(base, COMB_PER_W)])


def _add_kernel(g0_ref, g1_ref, bdec_ref, out_ref):
    out_ref[...] = g0_ref[...] + g1_ref[...] + bdec_ref[...]


@jax.jit
def kernel(x, enc_W, enc_b, dec_W, gater_W, gater_b, b_dec, b_gate):
    idx2, w2 = pl.pallas_call(
        _gate_kernel,
        out_shape=[jax.ShapeDtypeStruct((N_TOK, 2), jnp.int32),
                   jax.ShapeDtypeStruct((N_TOK, 2), jnp.float32)],
    )(x, gater_W, gater_b.reshape(1, EXPERTS), b_gate.reshape(1, ACT_DIM))

    pe = jnp.concatenate([idx2[:, 0], idx2[:, 1]])
    pw = jnp.concatenate([w2[:, 0], w2[:, 1]])
    valid = pw > 0.0
    oh = ((pe[:, None] == jnp.arange(EXPERTS)[None, :]) & valid[:, None])
    ohi = oh.astype(jnp.int32)
    counts = jnp.sum(ohi, axis=0)
    rank = jnp.sum((jnp.cumsum(ohi, axis=0) - ohi) * ohi, axis=1)
    tiles_e = (counts + TP - 1) // TP
    tile_end = jnp.cumsum(tiles_e)
    pad_start = (tile_end - tiles_e) * TP
    pos = jnp.where(valid, pad_start[pe] + rank, P_PAD)
    tok = jnp.concatenate([jnp.arange(N_TOK, dtype=jnp.int32)] * 2)
    tok_pad = jnp.zeros((P_PAD,), jnp.int32).at[pos].set(tok, mode="drop")
    w_pad = jnp.zeros((P_PAD, 1), jnp.float32).at[pos, 0].set(pw, mode="drop")
    used = tile_end[EXPERTS - 1]
    ids = jnp.arange(NT)
    eot_raw = jnp.clip(jnp.searchsorted(tile_end, ids, side="right"),
                       0, EXPERTS - 1)
    last_e = eot_raw[jnp.clip(used - 1, 0, NT - 1)]
    eot = jnp.where(ids < used, eot_raw, last_e).astype(jnp.int32)
    hw = (ids < used).astype(jnp.int32)
    sent = P_PAD - 1
    pos0 = jnp.where(valid[:N_TOK], pos[:N_TOK], sent).astype(jnp.int32)
    pos1 = jnp.where(valid[N_TOK:], pos[N_TOK:], sent).astype(jnp.int32)

    mesh = plsc.VectorSubcoreMesh(core_axis_name="c", subcore_axis_name="s")
    xs = pl.kernel(
        _sc_dispatch, mesh=mesh,
        out_type=jax.ShapeDtypeStruct((P_PAD, ACT_DIM), jnp.float32),
        scratch_types=[pltpu.VMEM((DISP_PER_W,), jnp.int32),
                       pltpu.VMEM((DISP_PER_W, ACT_DIM), jnp.float32),
                       pltpu.SemaphoreType.DMA],
    )(x, tok_pad)

    outp = pl.pallas_call(
        _grouped_kernel,
        grid_spec=pltpu.PrefetchScalarGridSpec(
            num_scalar_prefetch=2,
            grid=(NT,),
            in_specs=[
                pl.BlockSpec((TP, ACT_DIM), lambda i, eot, hw: (i, 0)),
                pl.BlockSpec((1, EXPERT_DICT, ACT_DIM),
                             lambda i, eot, hw: (eot[i], 0, 0)),
                pl.BlockSpec((1, 1, EXPERT_DICT),
                             lambda i, eot, hw: (eot[i], 0, 0)),
                pl.BlockSpec((1, EXPERT_DICT, ACT_DIM),
                             lambda i, eot, hw: (eot[i], 0, 0)),
                pl.BlockSpec((TP, 1), lambda i, eot, hw: (i, 0)),
                pl.BlockSpec((1, ACT_DIM), lambda i, eot, hw: (0, 0)),
            ],
            out_specs=pl.BlockSpec((TP, ACT_DIM), lambda i, eot, hw: (i, 0)),
        ),
        out_shape=jax.ShapeDtypeStruct((P_PAD, ACT_DIM), jnp.float32),
    )(eot, hw, xs, enc_W, enc_b.reshape(EXPERTS, 1, EXPERT_DICT), dec_W,
      w_pad, b_dec.reshape(1, ACT_DIM))

    g0, g1 = pl.kernel(
        _sc_combine, mesh=mesh,
        out_type=[jax.ShapeDtypeStruct((N_TOK, ACT_DIM), jnp.float32),
                  jax.ShapeDtypeStruct((N_TOK, ACT_DIM), jnp.float32)],
        scratch_types=[pltpu.VMEM((COMB_PER_W,), jnp.int32),
                       pltpu.VMEM((COMB_PER_W,), jnp.int32),
                       pltpu.VMEM((COMB_PER_W, ACT_DIM), jnp.float32),
                       pltpu.VMEM((COMB_PER_W, ACT_DIM), jnp.float32),
                       pltpu.SemaphoreType.DMA],
    )(outp, pos0, pos1)

    return pl.pallas_call(
        _add_kernel,
        grid=(N_TOK // 256,),
        in_specs=[pl.BlockSpec((256, ACT_DIM), lambda t: (t, 0)),
                  pl.BlockSpec((256, ACT_DIM), lambda t: (t, 0)),
                  pl.BlockSpec((1, ACT_DIM), lambda t: (0, 0))],
        out_specs=pl.BlockSpec((256, ACT_DIM), lambda t: (t, 0)),
        out_shape=jax.ShapeDtypeStruct((N_TOK, ACT_DIM), jnp.float32),
    )(g0, g1, b_dec.reshape(1, ACT_DIM))


# speedup vs baseline: 32.1263x; 1.2345x over previous
import functools

import jax
import jax.numpy as jnp
from jax import lax
from jax.experimental import pallas as pl
from jax.experimental.pallas import tpu as pltpu
from jax.experimental.pallas import tpu_sc as plsc

ACT_DIM = 768
EXPERTS = 16
K = 32
EXPERT_DICT = 1536
N_TOK = 2048
TP = 64
N_PAIRS = 2 * N_TOK
NT = 84
P_PAD = NT * TP
NW = 32
COMB_PER_W = N_TOK // NW


def _gate_kernel(x_ref, gaterW_ref, gaterb_ref, bgate_ref, idx_ref, w_ref):
    x = x_ref[...]
    n = x.shape[0]
    col = lax.broadcasted_iota(jnp.int32, (n, EXPERTS), 1)
    gl = lax.dot_general(x - bgate_ref[...], gaterW_ref[...],
                         (((1,), (1,)), ((), ())),
                         preferred_element_type=jnp.float32) + gaterb_ref[...]
    probs = jnp.maximum(gl, 0.0)
    m1 = jnp.max(probs, axis=1, keepdims=True)
    i1 = jnp.min(jnp.where(probs == m1, col, EXPERTS), axis=1, keepdims=True)
    probs2 = jnp.where(col == i1, -1.0, probs)
    m2 = jnp.max(probs2, axis=1, keepdims=True)
    i2 = jnp.min(jnp.where(probs2 == m2, col, EXPERTS), axis=1, keepdims=True)
    in_top0 = ((i1 == 0) | (i2 == 0)).astype(jnp.float32)
    in_top1 = ((i1 == 1) | (i2 == 1)).astype(jnp.float32)
    idx_ref[...] = jnp.concatenate([i1, i2], axis=1)
    w_ref[...] = jnp.concatenate([probs[:, 0:1] * in_top0,
                                  probs[:, 1:2] * in_top1], axis=1)


def _topk_mask(z):
    zi = lax.bitcast_convert_type(z, jnp.int32)
    rows = z.shape[0]
    t = jnp.zeros((rows, 1), jnp.int32)
    for bit in range(30, -1, -1):
        cand = t | (1 << bit)
        cnt = jnp.sum((zi >= cand).astype(jnp.int32), axis=1, keepdims=True)
        t = jnp.where(cnt >= K, cand, t)
    gt = zi > t
    n_gt = jnp.sum(gt.astype(jnp.int32), axis=1, keepdims=True)
    m = K - n_gt
    eq = zi == t
    lane = lax.broadcasted_iota(jnp.int32, z.shape, 1)
    c = jnp.zeros((rows, 1), jnp.int32)
    for bit in range(10, -1, -1):
        cand = c | (1 << bit)
        f = jnp.sum((eq & (lane < cand)).astype(jnp.int32), axis=1,
                    keepdims=True)
        c = jnp.where(f < m, cand, c)
    return gt | (eq & (lane <= c))


def _grouped_kernel(eot_ref, hw_ref, tok_ref, x_ref, encW_ref, encb_ref,
                    decW_ref, wp_ref, bdec_ref, outp_ref):
    i = pl.program_id(0)

    @pl.when(hw_ref[i] == 0)
    def _zero():
        outp_ref[...] = jnp.zeros_like(outp_ref)

    @pl.when(hw_ref[i] != 0)
    def _work():
        oh = (tok_ref[...] == lax.broadcasted_iota(
            jnp.int32, (TP, N_TOK), 1)).astype(jnp.float32)
        xg = lax.dot_general(oh, x_ref[...], (((1,), (0,)), ((), ())),
                             preferred_element_type=jnp.float32)
        xc = xg - bdec_ref[...]
        z = jnp.maximum(
            lax.dot_general(xc, encW_ref[0], (((1,), (1,)), ((), ())),
                            preferred_element_type=jnp.float32)
            + encb_ref[0], 0.0)
        mz = jnp.where(_topk_mask(z), z, 0.0)
        outp_ref[...] = lax.dot_general(
            mz, decW_ref[0], (((1,), (0,)), ((), ())),
            preferred_element_type=jnp.float32) * wp_ref[...]


def _sc_combine(outp_hbm, pos0_hbm, pos1_hbm, g0_hbm, g1_hbm,
                p0_v, p1_v, r0_v, r1_v, sem):
    wid = lax.axis_index("s") * 2 + lax.axis_index("c")
    base = wid * COMB_PER_W
    pltpu.sync_copy(pos0_hbm.at[pl.ds(base, COMB_PER_W)], p0_v)
    pltpu.sync_copy(pos1_hbm.at[pl.ds(base, COMB_PER_W)], p1_v)
    c0 = pltpu.async_copy(outp_hbm.at[p0_v], r0_v, sem)
    c1 = pltpu.async_copy(outp_hbm.at[p1_v], r1_v, sem)
    c0.wait()
    c1.wait()
    pltpu.sync_copy(r0_v, g0_hbm.at[pl.ds(base, COMB_PER_W)])
    pltpu.sync_copy(r1_v, g1_hbm.at[pl.ds(base, COMB_PER_W)])


def _add_kernel(g0_ref, g1_ref, bdec_ref, out_ref):
    out_ref[...] = g0_ref[...] + g1_ref[...] + bdec_ref[...]


@jax.jit
def kernel(x, enc_W, enc_b, dec_W, gater_W, gater_b, b_dec, b_gate):
    idx2, w2 = pl.pallas_call(
        _gate_kernel,
        out_shape=[jax.ShapeDtypeStruct((N_TOK, 2), jnp.int32),
                   jax.ShapeDtypeStruct((N_TOK, 2), jnp.float32)],
    )(x, gater_W, gater_b.reshape(1, EXPERTS), b_gate.reshape(1, ACT_DIM))

    pe = jnp.concatenate([idx2[:, 0], idx2[:, 1]])
    pw = jnp.concatenate([w2[:, 0], w2[:, 1]])
    valid = pw > 0.0
    oh = ((pe[:, None] == jnp.arange(EXPERTS)[None, :]) & valid[:, None])
    ohi = oh.astype(jnp.int32)
    counts = jnp.sum(ohi, axis=0)
    rank = jnp.sum((jnp.cumsum(ohi, axis=0) - ohi) * ohi, axis=1)
    tiles_e = (counts + TP - 1) // TP
    tile_end = jnp.cumsum(tiles_e)
    pad_start = (tile_end - tiles_e) * TP
    pos = jnp.where(valid, pad_start[pe] + rank, P_PAD)
    tok = jnp.concatenate([jnp.arange(N_TOK, dtype=jnp.int32)] * 2)
    tok_pad = jnp.zeros((P_PAD, 1), jnp.int32).at[pos, 0].set(tok, mode="drop")
    w_pad = jnp.zeros((P_PAD, 1), jnp.float32).at[pos, 0].set(pw, mode="drop")
    used = tile_end[EXPERTS - 1]
    ids = jnp.arange(NT)
    eot_raw = jnp.clip(jnp.searchsorted(tile_end, ids, side="right"),
                       0, EXPERTS - 1)
    last_e = eot_raw[jnp.clip(used - 1, 0, NT - 1)]
    eot = jnp.where(ids < used, eot_raw, last_e).astype(jnp.int32)
    hw = (ids < used).astype(jnp.int32)
    sent = P_PAD - 1
    pos0 = jnp.where(valid[:N_TOK], pos[:N_TOK], sent).astype(jnp.int32)
    pos1 = jnp.where(valid[N_TOK:], pos[N_TOK:], sent).astype(jnp.int32)

    outp = pl.pallas_call(
        _grouped_kernel,
        grid_spec=pltpu.PrefetchScalarGridSpec(
            num_scalar_prefetch=2,
            grid=(NT,),
            in_specs=[
                pl.BlockSpec((TP, 1), lambda i, eot, hw: (i, 0)),
                pl.BlockSpec((N_TOK, ACT_DIM), lambda i, eot, hw: (0, 0)),
                pl.BlockSpec((1, EXPERT_DICT, ACT_DIM),
                             lambda i, eot, hw: (eot[i], 0, 0)),
                pl.BlockSpec((1, 1, EXPERT_DICT),
                             lambda i, eot, hw: (eot[i], 0, 0)),
                pl.BlockSpec((1, EXPERT_DICT, ACT_DIM),
                             lambda i, eot, hw: (eot[i], 0, 0)),
                pl.BlockSpec((TP, 1), lambda i, eot, hw: (i, 0)),
                pl.BlockSpec((1, ACT_DIM), lambda i, eot, hw: (0, 0)),
            ],
            out_specs=pl.BlockSpec((TP, ACT_DIM), lambda i, eot, hw: (i, 0)),
        ),
        out_shape=jax.ShapeDtypeStruct((P_PAD, ACT_DIM), jnp.float32),
    )(eot, hw, tok_pad, x, enc_W, enc_b.reshape(EXPERTS, 1, EXPERT_DICT),
      dec_W, w_pad, b_dec.reshape(1, ACT_DIM))

    mesh = plsc.VectorSubcoreMesh(core_axis_name="c", subcore_axis_name="s")
    g0, g1 = pl.kernel(
        _sc_combine, mesh=mesh,
        out_type=[jax.ShapeDtypeStruct((N_TOK, ACT_DIM), jnp.float32),
                  jax.ShapeDtypeStruct((N_TOK, ACT_DIM), jnp.float32)],
        scratch_types=[pltpu.VMEM((COMB_PER_W,), jnp.int32),
                       pltpu.VMEM((COMB_PER_W,), jnp.int32),
                       pltpu.VMEM((COMB_PER_W, ACT_DIM), jnp.float32),
                       pltpu.VMEM((COMB_PER_W, ACT_DIM), jnp.float32),
                       pltpu.SemaphoreType.DMA],
    )(outp, pos0, pos1)

    return pl.pallas_call(
        _add_kernel,
        grid=(N_TOK // 256,),
        in_specs=[pl.BlockSpec((256, ACT_DIM), lambda t: (t, 0)),
                  pl.BlockSpec((256, ACT_DIM), lambda t: (t, 0)),
                  pl.BlockSpec((1, ACT_DIM), lambda t: (0, 0))],
        out_specs=pl.BlockSpec((256, ACT_DIM), lambda t: (t, 0)),
        out_shape=jax.ShapeDtypeStruct((N_TOK, ACT_DIM), jnp.float32),
    )(g0, g1, b_dec.reshape(1, ACT_DIM))
